# baseline (speedup 1.0000x reference)
import functools

import jax
import jax.numpy as jnp
from jax import lax
from jax.experimental import pallas as pl
from jax.experimental.pallas import tpu as pltpu
from jax.experimental.pallas import tpu_sc as plsc

_TARGET = 0
_TEST = 0.0

_ROWS = 32768
_COLS = 256
_NC = 2
_NS = 16
_L = 16
_NW = _NC * _NS
_ROWS_PER_W = _ROWS // _NW
_CHUNK = 128


def _join_col0(c0, d0):
    lo = c0 - d0
    hi = c0 + d0
    left = lo <= _TEST
    right = hi > _TEST
    min_hi = jnp.minimum(hi, _TEST)
    cl = (lo + min_hi) * 0.5
    dl = (min_hi - lo) * 0.5
    max_lo = jnp.maximum(lo, _TEST)
    cr = (max_lo + hi) * 0.5
    dr = (hi - max_lo) * 0.5
    both = left & right
    lj = jnp.minimum(cl - dl, cr - dr)
    rj = jnp.maximum(cl + dl, cr + dr)
    cb = (lj + rj) * 0.5
    db = (rj - lj) * 0.5
    new_c0 = jnp.where(both, cb, jnp.where(left, cl, cr))
    new_d0 = jnp.where(both, db, jnp.where(left, dl, dr))
    return new_c0, new_d0


def _sc_body(c_hbm, d_hbm, oc_hbm, od_hbm, cvm, dvm):
    wid = lax.axis_index("s") * _NC + lax.axis_index("c")
    base = wid * (_ROWS_PER_W * _COLS)
    iota = lax.iota(jnp.int32, _L)
    for k in range(_ROWS_PER_W // _CHUNK):
        e0 = base + k * (_CHUNK * _COLS)
        pltpu.sync_copy(c_hbm.at[pl.ds(e0, _CHUNK * _COLS)], cvm)
        pltpu.sync_copy(d_hbm.at[pl.ds(e0, _CHUNK * _COLS)], dvm)
        for j in range(_CHUNK // _L):
            eidx = (iota + (j * _L)) * _COLS + _TARGET
            c0 = plsc.load_gather(cvm, [eidx])
            d0 = plsc.load_gather(dvm, [eidx])
            new_c0, new_d0 = _join_col0(c0, d0)
            plsc.store_scatter(cvm, [eidx], new_c0)
            plsc.store_scatter(dvm, [eidx], new_d0)
        pltpu.sync_copy(cvm, oc_hbm.at[pl.ds(e0, _CHUNK * _COLS)])
        pltpu.sync_copy(dvm, od_hbm.at[pl.ds(e0, _CHUNK * _COLS)])


def kernel(c, delta, idx):
    del idx
    mesh = plsc.VectorSubcoreMesh(core_axis_name="c", subcore_axis_name="s")
    f = functools.partial(
        pl.kernel,
        out_type=[
            jax.ShapeDtypeStruct((_ROWS * _COLS,), jnp.float32),
            jax.ShapeDtypeStruct((_ROWS * _COLS,), jnp.float32),
        ],
        mesh=mesh,
        scratch_types=[
            pltpu.VMEM((_CHUNK * _COLS,), jnp.float32),
            pltpu.VMEM((_CHUNK * _COLS,), jnp.float32),
        ],
        compiler_params=pltpu.CompilerParams(needs_layout_passes=False),
    )(_sc_body)
    oc, od = f(c.reshape(-1), delta.reshape(-1))
    return oc.reshape(_ROWS, _COLS), od.reshape(_ROWS, _COLS)


# speedup vs baseline: 1.3740x; 1.3740x over previous
import functools

import jax
import jax.numpy as jnp
from jax import lax
from jax.experimental import pallas as pl
from jax.experimental.pallas import tpu as pltpu
from jax.experimental.pallas import tpu_sc as plsc

_TARGET = 0
_TEST = 0.0

_ROWS = 32768
_COLS = 256
_NC = 2
_NS = 16
_L = 16
_NW = _NC * _NS
_ROWS_PER_W = _ROWS // _NW
_SLAB = 128

_BLOCK_ROWS = 4096


def _join_col0(c0, d0):
    lo = c0 - d0
    hi = c0 + d0
    left = lo <= _TEST
    right = hi > _TEST
    min_hi = jnp.minimum(hi, _TEST)
    cl = (lo + min_hi) * 0.5
    dl = (min_hi - lo) * 0.5
    max_lo = jnp.maximum(lo, _TEST)
    cr = (max_lo + hi) * 0.5
    dr = (hi - max_lo) * 0.5
    both = left & right
    lj = jnp.minimum(cl - dl, cr - dr)
    rj = jnp.maximum(cl + dl, cr + dr)
    cb = (lj + rj) * 0.5
    db = (rj - lj) * 0.5
    new_c0 = jnp.where(both, cb, jnp.where(left, cl, cr))
    new_d0 = jnp.where(both, db, jnp.where(left, dl, dr))
    return new_c0, new_d0


def _sc_body(c_hbm, d_hbm, nc_hbm, nd_hbm, idxv, cg, dg, ocv, odv, sem):
    wid = lax.axis_index("s") * _NC + lax.axis_index("c")
    rbase = wid * _ROWS_PER_W
    iota = lax.iota(jnp.int32, _L)
    nslab = _ROWS_PER_W // _SLAB
    for k in range(nslab):
        for j in range(_SLAB // _L):
            ridx = iota + (rbase + k * _SLAB + j * _L)
            idxv[k, pl.ds(j * _L, _L)] = ridx * _COLS + _TARGET
    for k in range(nslab):
        pltpu.async_copy(c_hbm.at[idxv.at[k]], cg.at[k], sem)
        pltpu.async_copy(d_hbm.at[idxv.at[k]], dg.at[k], sem)
    for k in range(nslab):
        pltpu.make_async_copy(c_hbm.at[idxv.at[k]], cg.at[k], sem).wait()
        pltpu.make_async_copy(d_hbm.at[idxv.at[k]], dg.at[k], sem).wait()
    for k in range(nslab):
        for j in range(_SLAB // _L):
            c0 = cg[k, pl.ds(j * _L, _L)]
            d0 = dg[k, pl.ds(j * _L, _L)]
            new_c0, new_d0 = _join_col0(c0, d0)
            ocv[pl.ds(k * _SLAB + j * _L, _L)] = new_c0
            odv[pl.ds(k * _SLAB + j * _L, _L)] = new_d0
    pltpu.sync_copy(ocv, nc_hbm.at[pl.ds(rbase, _ROWS_PER_W)])
    pltpu.sync_copy(odv, nd_hbm.at[pl.ds(rbase, _ROWS_PER_W)])


def _sc_col_join(c, delta):
    mesh = plsc.VectorSubcoreMesh(core_axis_name="c", subcore_axis_name="s")
    f = functools.partial(
        pl.kernel,
        out_type=[
            jax.ShapeDtypeStruct((_ROWS,), jnp.float32),
            jax.ShapeDtypeStruct((_ROWS,), jnp.float32),
        ],
        mesh=mesh,
        scratch_types=[
            pltpu.VMEM((_ROWS_PER_W // _SLAB, _SLAB), jnp.int32),
            pltpu.VMEM((_ROWS_PER_W // _SLAB, _SLAB), jnp.float32),
            pltpu.VMEM((_ROWS_PER_W // _SLAB, _SLAB), jnp.float32),
            pltpu.VMEM((_ROWS_PER_W,), jnp.float32),
            pltpu.VMEM((_ROWS_PER_W,), jnp.float32),
            pltpu.SemaphoreType.DMA,
        ],
        compiler_params=pltpu.CompilerParams(needs_layout_passes=False),
    )(_sc_body)
    return f(c.reshape(-1), delta.reshape(-1))


def _tc_patch(c_ref, d_ref, nc_ref, nd_ref, oc_ref, od_ref):
    c = c_ref[...]
    d = d_ref[...]
    nc = nc_ref[...]
    nd = nd_ref[...]
    col = jax.lax.broadcasted_iota(jnp.int32, c.shape, 1)
    is_t = col == _TARGET
    oc_ref[...] = jnp.where(is_t, nc, c)
    od_ref[...] = jnp.where(is_t, nd, d)


def kernel(c, delta, idx):
    del idx
    ncol, dcol = _sc_col_join(c, delta)
    nc2 = ncol.reshape(_ROWS, 1)
    nd2 = dcol.reshape(_ROWS, 1)
    spec = pl.BlockSpec((_BLOCK_ROWS, _COLS), lambda i: (i, 0))
    vspec = pl.BlockSpec((_BLOCK_ROWS, 1), lambda i: (i, 0))
    out_c, out_d = pl.pallas_call(
        _tc_patch,
        grid=(_ROWS // _BLOCK_ROWS,),
        in_specs=[spec, spec, vspec, vspec],
        out_specs=[spec, spec],
        out_shape=[
            jax.ShapeDtypeStruct((_ROWS, _COLS), jnp.float32),
            jax.ShapeDtypeStruct((_ROWS, _COLS), jnp.float32),
        ],
        compiler_params=pltpu.CompilerParams(
            dimension_semantics=("parallel",),
        ),
    )(c, delta, nc2, nd2)
    return out_c, out_d
